# baseline (speedup 1.0000x reference)
import functools

import jax
import jax.numpy as jnp
from jax import lax
from jax.experimental import pallas as pl
from jax.experimental.pallas import tpu as pltpu
from jax.experimental.pallas import tpu_sc as plsc


def _lse_body(w_ref, lse_ref):
    w = w_ref[...]
    m = jnp.max(w, axis=1)
    lse_ref[...] = m + jnp.log(jnp.sum(jnp.exp(w - m[:, None]), axis=1))


def _row_lse(W):
    V = W.shape[0]
    return pl.pallas_call(
        _lse_body,
        out_shape=jax.ShapeDtypeStruct((V,), jnp.float32),
    )(W)


_LANES = 16


_LPAD = 64
_SPAD = 56
_VPAD = 1024


def _sc_gather(idx2, tgt2, Wp, lse, *, B, L, nw):
    V = Wp.shape[0]
    rows = B // nw
    mesh = plsc.VectorSubcoreMesh(core_axis_name="c", subcore_axis_name="s")
    info = plsc.get_sparse_core_info()
    nc = info.num_cores
    groups = (L + _LANES - 1) // _LANES

    @functools.partial(
        pl.kernel,
        mesh=mesh,
        compiler_params=pltpu.CompilerParams(
            use_tc_tiling_on_sc=True, needs_layout_passes=False),
        out_type=[
            jax.ShapeDtypeStruct((B, _SPAD, _VPAD), jnp.float32),
            jax.ShapeDtypeStruct((nw * 1024,), jnp.float32),
            jax.ShapeDtypeStruct((nw * 1024,), jnp.float32),
        ],
        scratch_types=[
            pltpu.VMEM((rows * _LPAD,), jnp.int32),
            pltpu.VMEM((rows * _LPAD,), jnp.int32),
            pltpu.VMEM((_VPAD,), jnp.float32),
            pltpu.VMEM((_SPAD, 8, 128), jnp.float32),
            pltpu.VMEM((_SPAD, 8, 128), jnp.float32),
            pltpu.VMEM((1024,), jnp.float32),
            pltpu.VMEM((1024,), jnp.float32),
            pltpu.SemaphoreType.DMA,
            pltpu.SemaphoreType.DMA,
            pltpu.SemaphoreType.DMA,
            pltpu.SemaphoreType.DMA,
        ],
    )
    def k(idx_hbm, tgt_hbm, w_hbm, lse_hbm, out_hbm, acc_hbm, cnt_hbm,
          idx_v, tgt_v, lse_v, buf0, buf1, acc_s, cnt_s,
          gsem0, gsem1, ssem0, ssem1):
        wid = lax.axis_index("s") * nc + lax.axis_index("c")
        nstg = rows * _LPAD
        bufs = (buf0, buf1)
        gsems = (gsem0, gsem1)
        ssems = (ssem0, ssem1)
        pltpu.sync_copy(idx_hbm.at[pl.ds(wid * nstg, nstg)], idx_v)
        pltpu.sync_copy(tgt_hbm.at[pl.ds(wid * nstg, nstg)], tgt_v)
        pltpu.sync_copy(lse_hbm, lse_v)
        acc_v = jnp.zeros((_LANES,), jnp.float32)
        cnt_v = jnp.zeros((_LANES,), jnp.float32)

        def gather(g, buf, sem):
            ilist = idx_v.at[pl.ds(g * _LPAD, L)]
            return pltpu.make_async_copy(
                w_hbm.at[ilist], buf.at[pl.ds(0, L)], sem)

        def store_one(g, buf, c, sem):
            b = wid * rows + g
            return pltpu.make_async_copy(
                buf.at[:, c], out_hbm.at[b, :, pl.ds(c * 128, 128)], sem)

        def store_all(g, buf, sem):
            for c in range(8):
                store_one(g, buf, c, sem).start()

        def store_wait(g, buf, sem):
            for c in range(8):
                store_one(g, buf, c, sem).wait()

        gather(0, buf0, gsem0).start()
        gather(1, buf1, gsem1).start()

        def step(i, carries):
            acc_c, cnt_c = carries
            for par in range(2):
                g = 2 * i + par
                buf, gsem, ssem = bufs[par], gsems[par], ssems[par]
                gather(g, buf, gsem).wait()
                store_all(g, buf, ssem)
                for j in range(groups):
                    sl = pl.ds(g * _LPAD + j * _LANES, _LANES)
                    lanes = jnp.arange(_LANES, dtype=jnp.int32) + j * _LANES
                    i16 = jnp.clip(idx_v[sl], 0, V - 1)
                    t16 = tgt_v[sl]
                    m = t16 != -1
                    tsafe = jnp.where(m, t16, 0)
                    row16 = jnp.minimum(lanes, L - 1)
                    wit = plsc.load_gather(
                        buf, [row16, tsafe >> 7, tsafe & 127])
                    ls16 = plsc.load_gather(lse_v, [i16])
                    acc_c = acc_c + jnp.where(m, wit - ls16, 0.0)
                    cnt_c = cnt_c + jnp.where(m, 1.0, 0.0)

                @pl.when(g + 2 < rows)
                def _refill():
                    store_wait(g, buf, ssem)
                    gather(g + 2, buf, gsem).start()

            return (acc_c, cnt_c)

        acc_v, cnt_v = lax.fori_loop(0, rows // 2, step, (acc_v, cnt_v))
        store_wait(rows - 2, buf0, ssem0)
        store_wait(rows - 1, buf1, ssem1)
        for j in range(1024 // _LANES):
            z = jnp.zeros((_LANES,), jnp.float32)
            acc_s[pl.ds(j * _LANES, _LANES)] = z
            cnt_s[pl.ds(j * _LANES, _LANES)] = z
        acc_s[pl.ds(0, _LANES)] = acc_v
        cnt_s[pl.ds(0, _LANES)] = cnt_v
        pltpu.sync_copy(acc_s, acc_hbm.at[pl.ds(wid * 1024, 1024)])
        pltpu.sync_copy(cnt_s, cnt_hbm.at[pl.ds(wid * 1024, 1024)])

    return k(idx2, tgt2, Wp, lse)


def _fin_body(acc_ref, cnt_ref, out_ref):
    s = jnp.sum(acc_ref[...])
    c = jnp.sum(cnt_ref[...])
    out_ref[...] = jnp.full((1, 1), -(s / jnp.maximum(c, 1.0)), jnp.float32)


def _finalize(acc, cnt):
    return pl.pallas_call(
        _fin_body,
        out_shape=jax.ShapeDtypeStruct((1, 1), jnp.float32),
    )(acc, cnt)


def kernel(idx, targets, W):
    B, L = idx.shape
    V = W.shape[0]
    info = plsc.get_sparse_core_info()
    nw = info.num_cores * info.num_subcores
    assert B % (2 * nw) == 0

    rows = B // nw
    idx2 = jnp.pad(idx.astype(jnp.int32), ((0, 0), (0, _LPAD - L))
                   ).reshape(nw * rows * _LPAD)
    tgt2 = jnp.pad(targets.astype(jnp.int32), ((0, 0), (0, _LPAD - L)),
                   constant_values=-1).reshape(nw * rows * _LPAD)
    Wp = jnp.pad(W, ((0, 0), (0, _VPAD - V))).reshape(V, 8, 128)
    lse = jnp.pad(_row_lse(W), (0, _VPAD - V))
    raw, acc, cnt = _sc_gather(idx2, tgt2, Wp, lse, B=B, L=L, nw=nw)
    logits = raw[:, :L, :V]
    loss = _finalize(acc, cnt)[0, 0]
    return logits, loss
